# baseline (speedup 1.0000x reference)
import functools

import jax
import jax.numpy as jnp
from jax import lax
from jax.experimental import pallas as pl
from jax.experimental.pallas import tpu as pltpu
from jax.experimental.pallas import tpu_sc as plsc

N_CORES = 2
N_SUBCORES = 16
NW = N_CORES * N_SUBCORES
LANES = 16

BATCH = 16384
DIM = 32
B_PER_W = BATCH // NW
CHUNK = 128
N_CHUNK = B_PER_W // CHUNK
GROUPS = CHUNK // LANES


def _mf_body(users_hbm, items_hbm, ul_hbm, il_hbm, ub_hbm, ib_hbm, out_hbm,
             idx_u, idx_i, u_rows, i_rows, ub_v, ib_v, out_v, sem):
    wid = lax.axis_index("s") * N_CORES + lax.axis_index("c")

    pltpu.sync_copy(users_hbm.at[wid], idx_u)
    pltpu.sync_copy(items_hbm.at[wid], idx_i)

    copies = []
    for j in range(N_CHUNK):
        copies.append(pltpu.async_copy(ul_hbm.at[idx_u.at[j]], u_rows.at[j], sem))
        copies.append(pltpu.async_copy(il_hbm.at[idx_i.at[j]], i_rows.at[j], sem))
        copies.append(pltpu.async_copy(ub_hbm.at[idx_u.at[j]], ub_v.at[j], sem))
        copies.append(pltpu.async_copy(ib_hbm.at[idx_i.at[j]], ib_v.at[j], sem))
    for c in copies:
        c.wait()

    lane = lax.broadcasted_iota(jnp.int32, (LANES,), 0)

    for j in range(N_CHUNK):
        def group_body(g, _, j=j):
            acc = ub_v[j, pl.ds(g * LANES, LANES)] + ib_v[j, pl.ds(g * LANES, LANES)]
            for r in range(LANES):
                row = g * LANES + r
                u0 = u_rows[j, row, pl.ds(0, LANES)]
                u1 = u_rows[j, row, pl.ds(LANES, LANES)]
                v0 = i_rows[j, row, pl.ds(0, LANES)]
                v1 = i_rows[j, row, pl.ds(LANES, LANES)]
                s = u0 * v0 + u1 * v1
                for shift in (8, 4, 2, 1):
                    s = s + s.at[lane ^ shift].get(mode="promise_in_bounds")
                acc = acc + jnp.where(lane == r, s, 0.0)
            out_v[j, pl.ds(g * LANES, LANES)] = acc
            return _

        lax.fori_loop(0, GROUPS, group_body, None)

    pltpu.sync_copy(out_v, out_hbm.at[wid])


_mf_kernel = functools.partial(
    pl.kernel,
    out_type=jax.ShapeDtypeStruct((NW, N_CHUNK, CHUNK), jnp.float32),
    mesh=plsc.VectorSubcoreMesh(core_axis_name="c", subcore_axis_name="s"),
    scratch_types=[
        pltpu.VMEM((N_CHUNK, CHUNK), jnp.int32),
        pltpu.VMEM((N_CHUNK, CHUNK), jnp.int32),
        pltpu.VMEM((N_CHUNK, CHUNK, DIM), jnp.float32),
        pltpu.VMEM((N_CHUNK, CHUNK, DIM), jnp.float32),
        pltpu.VMEM((N_CHUNK, CHUNK), jnp.float32),
        pltpu.VMEM((N_CHUNK, CHUNK), jnp.float32),
        pltpu.VMEM((N_CHUNK, CHUNK), jnp.float32),
        pltpu.SemaphoreType.DMA,
    ],
    compiler_params=pltpu.CompilerParams(
        use_tc_tiling_on_sc=False, needs_layout_passes=False),
)(_mf_body)


_RB = 1280
_NBLK = -(-1000000 // _RB)


def _repack_body(u_t_ref, i_t_ref, u_out_ref, i_out_ref):
    u_out_ref[...] = jnp.swapaxes(u_t_ref[...], 0, 1)
    i_out_ref[...] = jnp.swapaxes(i_t_ref[...], 0, 1)


def _repack(ul_t, il_t):
    return pl.pallas_call(
        _repack_body,
        grid=(_NBLK,),
        in_specs=[
            pl.BlockSpec((DIM, _RB), lambda j: (0, j)),
            pl.BlockSpec((DIM, _RB), lambda j: (0, j)),
        ],
        out_specs=[
            pl.BlockSpec((_RB, DIM), lambda j: (j, 0)),
            pl.BlockSpec((_RB, DIM), lambda j: (j, 0)),
        ],
        out_shape=[
            jax.ShapeDtypeStruct((1000000, DIM), jnp.float32),
            jax.ShapeDtypeStruct((1000000, DIM), jnp.float32),
        ],
    )(ul_t, il_t)


@jax.jit
def kernel(users, items, user_latent, item_latent, user_bias, item_bias):
    users_r = users.reshape(NW, N_CHUNK, CHUNK)
    items_r = items.reshape(NW, N_CHUNK, CHUNK)
    ul, il = _repack(user_latent.T, item_latent.T)
    ub = user_bias.reshape(-1)
    ib = item_bias.reshape(-1)
    out = _mf_kernel(users_r, items_r, ul, il, ub, ib)
    return out.reshape(BATCH)


# speedup vs baseline: 1.7240x; 1.7240x over previous
import functools

import jax
import jax.numpy as jnp
from jax import lax
from jax.experimental import pallas as pl
from jax.experimental.pallas import tpu as pltpu
from jax.experimental.pallas import tpu_sc as plsc

N_CORES = 2
N_SUBCORES = 16
NW = N_CORES * N_SUBCORES
LANES = 16

BATCH = 16384
DIM = 32
B_PER_W = BATCH // NW
CHUNK = 128
N_CHUNK = B_PER_W // CHUNK
GROUPS = CHUNK // LANES


def _mf_body(users_hbm, items_hbm, ul_hbm, il_hbm, ub_hbm, ib_hbm, out_hbm,
             idx_u, idx_i, u_rows, i_rows, ub_v, ib_v, out_v, sem):
    wid = lax.axis_index("s") * N_CORES + lax.axis_index("c")

    pltpu.sync_copy(users_hbm.at[wid], idx_u)
    pltpu.sync_copy(items_hbm.at[wid], idx_i)

    copies = []
    for j in range(N_CHUNK):
        copies.append(pltpu.async_copy(ul_hbm.at[idx_u.at[j]], u_rows.at[j], sem))
        copies.append(pltpu.async_copy(il_hbm.at[idx_i.at[j]], i_rows.at[j], sem))
        copies.append(pltpu.async_copy(ub_hbm.at[idx_u.at[j]], ub_v.at[j], sem))
        copies.append(pltpu.async_copy(ib_hbm.at[idx_i.at[j]], ib_v.at[j], sem))
    for c in copies:
        c.wait()

    lane = lax.broadcasted_iota(jnp.int32, (LANES,), 0)

    for j in range(N_CHUNK):
        def group_body(g, _, j=j):
            acc = ub_v[j, pl.ds(g * LANES, LANES)] + ib_v[j, pl.ds(g * LANES, LANES)]
            for r in range(LANES):
                row = g * LANES + r
                u0 = u_rows[j, row, pl.ds(0, LANES)]
                u1 = u_rows[j, row, pl.ds(LANES, LANES)]
                v0 = i_rows[j, row, pl.ds(0, LANES)]
                v1 = i_rows[j, row, pl.ds(LANES, LANES)]
                s = u0 * v0 + u1 * v1
                for shift in (8, 4, 2, 1):
                    s = s + s.at[lane ^ shift].get(mode="promise_in_bounds")
                acc = acc + jnp.where(lane == r, s, 0.0)
            out_v[j, pl.ds(g * LANES, LANES)] = acc
            return _

        lax.fori_loop(0, GROUPS, group_body, None)

    pltpu.sync_copy(out_v, out_hbm.at[wid])


_mf_kernel = functools.partial(
    pl.kernel,
    out_type=jax.ShapeDtypeStruct((NW, N_CHUNK, CHUNK), jnp.float32),
    mesh=plsc.VectorSubcoreMesh(core_axis_name="c", subcore_axis_name="s"),
    scratch_types=[
        pltpu.VMEM((N_CHUNK, CHUNK), jnp.int32),
        pltpu.VMEM((N_CHUNK, CHUNK), jnp.int32),
        pltpu.VMEM((N_CHUNK, CHUNK, DIM), jnp.float32),
        pltpu.VMEM((N_CHUNK, CHUNK, DIM), jnp.float32),
        pltpu.VMEM((N_CHUNK, CHUNK), jnp.float32),
        pltpu.VMEM((N_CHUNK, CHUNK), jnp.float32),
        pltpu.VMEM((N_CHUNK, CHUNK), jnp.float32),
        pltpu.SemaphoreType.DMA,
    ],
    compiler_params=pltpu.CompilerParams(use_tc_tiling_on_sc=False),
)(_mf_body)


@jax.jit
def kernel(users, items, user_latent, item_latent, user_bias, item_bias):
    users_r = users.reshape(NW, N_CHUNK, CHUNK)
    items_r = items.reshape(NW, N_CHUNK, CHUNK)
    ub = user_bias.reshape(-1)
    ib = item_bias.reshape(-1)
    out = _mf_kernel(users_r, items_r, user_latent, item_latent, ub, ib)
    return out.reshape(BATCH)
